# baseline (speedup 1.0000x reference)
import functools

import jax
import jax.numpy as jnp
from jax import lax
from jax.experimental import pallas as pl
from jax.experimental.pallas import tpu as pltpu
from jax.experimental.pallas import tpu_sc as plsc

N = 10000
E = 320000
NC = 2
NS = 16
NW = NC * NS
CHUNK = 128
CPW = 80
EPW = CPW * CHUNK
EPW_REAL = E // NW
NPS = 640
NPAD = NS * NPS


def _mesh():
    return plsc.VectorSubcoreMesh(
        core_axis_name="c", subcore_axis_name="s", num_cores=NC, num_subcores=NS
    )


_SC_PARAMS = pltpu.CompilerParams(
    needs_layout_passes=False, use_tc_tiling_on_sc=False
)

_GATHER_DNUMS = lax.GatherDimensionNumbers(
    offset_dims=(), collapsed_slice_dims=(0,), start_index_map=(0,)
)


def _bcast_lane(v, l):
    return lax.gather(
        v, jnp.full((16, 1), l, jnp.int32), _GATHER_DNUMS, (1,),
        mode=lax.GatherScatterMode.PROMISE_IN_BOUNDS,
    )


@functools.partial(
    pl.kernel,
    out_type=jax.ShapeDtypeStruct((NC, NPAD), jnp.float32),
    mesh=_mesh(),
    compiler_params=_SC_PARAMS,
    scratch_types=[
        pltpu.VMEM((CPW, CHUNK), jnp.int32),
        pltpu.VMEM((CPW, CHUNK), jnp.float32),
        pltpu.VMEM((NPS,), jnp.float32),
        pltpu.VMEM_SHARED((NPAD,), jnp.float32),
    ],
)
def _sc_deg(dst_hbm, ew_hbm, out_hbm, dst_v, ew_v, zb, acc):
    cid = lax.axis_index("c")
    sid = lax.axis_index("s")
    wid = cid * NS + sid

    def zero_zb(i, carry):
        zb[pl.ds(i * 16, 16)] = jnp.zeros((16,), jnp.float32)
        return carry

    lax.fori_loop(0, NPS // 16, zero_zb, 0)
    pltpu.sync_copy(zb, acc.at[pl.ds(sid * NPS, NPS)])
    plsc.subcore_barrier()

    pltpu.sync_copy(dst_hbm.at[wid], dst_v)
    pltpu.sync_copy(ew_hbm.at[wid], ew_v)

    def chunk(j, carry):
        pltpu.sync_copy(ew_v.at[j], acc.at[dst_v.at[j]], add=True)
        return carry

    lax.fori_loop(0, CPW, chunk, 0)
    plsc.subcore_barrier()
    pltpu.sync_copy(acc.at[pl.ds(sid * NPS, NPS)], zb)
    pltpu.sync_copy(zb, out_hbm.at[cid, pl.ds(sid * NPS, NPS)])


def _make_agg(D):
    @functools.partial(
        pl.kernel,
        out_type=jax.ShapeDtypeStruct((NC, NPAD, D), jnp.float32),
        mesh=_mesh(),
        compiler_params=_SC_PARAMS,
        scratch_types=[
            pltpu.VMEM((CPW, CHUNK), jnp.int32),
            pltpu.VMEM((CPW, CHUNK), jnp.int32),
            pltpu.VMEM((CPW, CHUNK), jnp.float32),
            pltpu.VMEM((NPAD,), jnp.float32),
            pltpu.VMEM((CHUNK, D), jnp.float32),
            pltpu.VMEM((CHUNK, D), jnp.float32),
            pltpu.VMEM((CHUNK, D), jnp.float32),
            pltpu.VMEM((CHUNK, D), jnp.float32),
            pltpu.VMEM((64, D), jnp.float32),
            pltpu.VMEM_SHARED((NPAD, D), jnp.float32),
            pltpu.SemaphoreType.DMA,
            pltpu.SemaphoreType.DMA,
            pltpu.SemaphoreType.DMA,
            pltpu.SemaphoreType.DMA,
            pltpu.SemaphoreType.DMA,
            pltpu.SemaphoreType.DMA,
            pltpu.SemaphoreType.DMA,
            pltpu.SemaphoreType.DMA,
        ],
    )
    def agg(h_hbm, src_hbm, dst_hbm, ew_hbm, dis_hbm, out_hbm,
            src_v, dst_v, ew_v, dis_v, r0, r1, r2, r3, zb, acc,
            g0, g1, g2, g3, s0, s1, s2, s3):
        cid = lax.axis_index("c")
        sid = lax.axis_index("s")
        wid = cid * NS + sid
        rows = [r0, r1, r2, r3]
        semg = [g0, g1, g2, g3]
        sems = [s0, s1, s2, s3]

        def zero_zb(r, carry):
            for c in range(D // 16):
                zb[r, pl.ds(c * 16, 16)] = jnp.zeros((16,), jnp.float32)
            return carry

        lax.fori_loop(0, 64, zero_zb, 0)

        def zero_acc(t, carry):
            pltpu.sync_copy(zb, acc.at[pl.ds(sid * NPS + t * 64, 64)])
            return carry

        lax.fori_loop(0, NPS // 64, zero_acc, 0)
        plsc.subcore_barrier()

        pltpu.sync_copy(src_hbm.at[wid], src_v)
        pltpu.sync_copy(dst_hbm.at[wid], dst_v)
        pltpu.sync_copy(ew_hbm.at[wid], ew_v)
        pltpu.sync_copy(dis_hbm, dis_v)

        def gather_start(j, k):
            pltpu.async_copy(h_hbm.at[src_v.at[j]], rows[k], semg[k])

        def gather_wait(j, k):
            pltpu.make_async_copy(h_hbm.at[src_v.at[j]], rows[k], semg[k]).wait()

        def scat_start(j, k):
            pltpu.async_copy(rows[k], acc.at[dst_v.at[j]], sems[k], add=True)

        def scat_wait(j, k):
            pltpu.make_async_copy(rows[k], acc.at[dst_v.at[j]], sems[k]).wait()

        def scale(j, k):
            norms = []
            for g in range(CHUNK // 16):
                si = src_v[j, pl.ds(g * 16, 16)]
                di = dst_v[j, pl.ds(g * 16, 16)]
                ev = ew_v[j, pl.ds(g * 16, 16)]
                norms.append(
                    plsc.load_gather(dis_v, [si]) * ev * plsc.load_gather(dis_v, [di])
                )
            for g in range(CHUNK // 16):
                for l in range(16):
                    r = g * 16 + l
                    b = _bcast_lane(norms[g], l)
                    for c in range(D // 16):
                        rows[k][r, pl.ds(c * 16, 16)] = (
                            rows[k][r, pl.ds(c * 16, 16)] * b
                        )

        gather_start(0, 0)
        gather_start(1, 1)

        NT = CPW // 4

        def body(t, carry):
            for k in range(4):
                c = t * 4 + k
                gather_wait(c, k)
                kn = (k + 2) % 4
                if k < 2:
                    @pl.when(t > 0)
                    def _():
                        scat_wait(c - 2, kn)

                    gather_start(c + 2, kn)
                else:
                    scat_wait(c - 2, kn)

                    @pl.when(t < NT - 1)
                    def _():
                        gather_start(c + 2, kn)
                scale(c, k)
                scat_start(c, k)
            return carry

        lax.fori_loop(0, NT, body, 0)
        scat_wait(CPW - 2, 2)
        scat_wait(CPW - 1, 3)
        plsc.subcore_barrier()

        def writeout(t, carry):
            pltpu.sync_copy(acc.at[pl.ds(sid * NPS + t * 64, 64)], zb)
            pltpu.sync_copy(zb, out_hbm.at[cid, pl.ds(sid * NPS + t * 64, 64)])
            return carry

        lax.fori_loop(0, NPS // 64, writeout, 0)

    return agg


_sc_agg64 = _make_agg(64)
_sc_agg32 = _make_agg(32)


def _mm1_body(x_ref, w_ref, b_ref, d_ref, o_ref, dis_ref):
    i = pl.program_id(0)
    o_ref[...] = (
        jnp.dot(x_ref[...], w_ref[...], preferred_element_type=jnp.float32, precision=lax.Precision.HIGHEST)
        + b_ref[...]
    )

    @pl.when(i == 0)
    def _():
        deg = d_ref[0:1, :] + d_ref[1:2, :] + 1.0
        dis_ref[...] = jnp.where(deg > 0, lax.rsqrt(deg), 0.0)


def _l2_body(a_ref, h1_ref, dc_ref, w2_ref, b1_ref, o_ref):
    s = dc_ref[...]
    out1 = a_ref[0] + a_ref[1] + s * s * h1_ref[...] + b1_ref[...]
    out1 = jnp.maximum(out1, 0.0)
    o_ref[...] = jnp.dot(out1, w2_ref[...], preferred_element_type=jnp.float32, precision=lax.Precision.HIGHEST)


def _zadj_body(ai_ref, hi_ref, di_ref, aj_ref, hj_ref, dj_ref, b2_ref,
               adj_ref, z_ref):
    j = pl.program_id(1)
    si = di_ref[...]
    zi = ai_ref[0] + ai_ref[1] + si * si * hi_ref[...] + b2_ref[...]
    sj = dj_ref[...]
    zj = aj_ref[0] + aj_ref[1] + sj * sj * hj_ref[...] + b2_ref[...]
    adj_ref[...] = lax.dot_general(
        zi, zj, (((1,), (1,)), ((), ())), preferred_element_type=jnp.float32
    )

    @pl.when(j == 0)
    def _():
        z_ref[...] = zi


def kernel(x, edge_index, edge_weight, W1, b1, W2, b2):
    src = edge_index[0].astype(jnp.int32)
    dst = edge_index[1].astype(jnp.int32)
    ew = edge_weight.astype(jnp.float32)
    pad = NW * EPW - E
    pad_idx = jnp.arange(pad, dtype=jnp.int32) % N
    src3 = jnp.concatenate([src, pad_idx]).reshape(NW, CPW, CHUNK)
    dst3 = jnp.concatenate([dst, pad_idx]).reshape(NW, CPW, CHUNK)
    ew3 = jnp.concatenate([ew, jnp.zeros((pad,), jnp.float32)]).reshape(
        NW, CPW, CHUNK
    )
    b1r = b1.reshape(1, 64)
    b2r = b2.reshape(1, 32)

    deg2 = _sc_deg(dst3, ew3)

    h1, dis2d = pl.pallas_call(
        _mm1_body,
        grid=(10,),
        in_specs=[
            pl.BlockSpec((1000, 128), lambda i: (i, 0)),
            pl.BlockSpec((128, 64), lambda i: (0, 0)),
            pl.BlockSpec((1, 64), lambda i: (0, 0)),
            pl.BlockSpec((NC, NPAD), lambda i: (0, 0)),
        ],
        out_specs=[
            pl.BlockSpec((1000, 64), lambda i: (i, 0)),
            pl.BlockSpec((1, NPAD), lambda i: (0, 0)),
        ],
        out_shape=[
            jax.ShapeDtypeStruct((N, 64), jnp.float32),
            jax.ShapeDtypeStruct((1, NPAD), jnp.float32),
        ],
    )(x, W1, b1r, deg2)
    dis_flat = dis2d.reshape(NPAD)
    dis_col = dis2d[0, :N].reshape(N, 1)

    agg1 = _sc_agg64(h1, src3, dst3, ew3, dis_flat)

    h2 = pl.pallas_call(
        _l2_body,
        grid=(10,),
        in_specs=[
            pl.BlockSpec((NC, 1000, 64), lambda i: (0, i, 0)),
            pl.BlockSpec((1000, 64), lambda i: (i, 0)),
            pl.BlockSpec((1000, 1), lambda i: (i, 0)),
            pl.BlockSpec((64, 32), lambda i: (0, 0)),
            pl.BlockSpec((1, 64), lambda i: (0, 0)),
        ],
        out_specs=pl.BlockSpec((1000, 32), lambda i: (i, 0)),
        out_shape=jax.ShapeDtypeStruct((N, 32), jnp.float32),
    )(agg1, h1, dis_col, W2, b1r)

    agg2 = _sc_agg32(h2, src3, dst3, ew3, dis_flat)

    adj, z = pl.pallas_call(
        _zadj_body,
        grid=(5, 8),
        in_specs=[
            pl.BlockSpec((NC, 2000, 32), lambda i, j: (0, i, 0)),
            pl.BlockSpec((2000, 32), lambda i, j: (i, 0)),
            pl.BlockSpec((2000, 1), lambda i, j: (i, 0)),
            pl.BlockSpec((NC, 1280, 32), lambda i, j: (0, j, 0)),
            pl.BlockSpec((1280, 32), lambda i, j: (j, 0)),
            pl.BlockSpec((1280, 1), lambda i, j: (j, 0)),
            pl.BlockSpec((1, 32), lambda i, j: (0, 0)),
        ],
        out_specs=[
            pl.BlockSpec((2000, 1280), lambda i, j: (i, j)),
            pl.BlockSpec((2000, 32), lambda i, j: (i, 0)),
        ],
        out_shape=[
            jax.ShapeDtypeStruct((N, N), jnp.float32),
            jax.ShapeDtypeStruct((N, 32), jnp.float32),
        ],
    )(agg2, h2, dis_col, agg2, h2, dis_col, b2r)

    return (adj, z)


# speedup vs baseline: 1.0224x; 1.0224x over previous
import functools

import jax
import jax.numpy as jnp
from jax import lax
from jax.experimental import pallas as pl
from jax.experimental.pallas import tpu as pltpu
from jax.experimental.pallas import tpu_sc as plsc

N = 10000
E = 320000
NC = 2
NS = 16
NW = NC * NS
CHUNK = 128
CPW = 80
EPW = CPW * CHUNK
EPW_REAL = E // NW
NPS = 640
NPAD = NS * NPS


def _mesh():
    return plsc.VectorSubcoreMesh(
        core_axis_name="c", subcore_axis_name="s", num_cores=NC, num_subcores=NS
    )


_SC_PARAMS = pltpu.CompilerParams(
    needs_layout_passes=False, use_tc_tiling_on_sc=False
)

_GATHER_DNUMS = lax.GatherDimensionNumbers(
    offset_dims=(), collapsed_slice_dims=(0,), start_index_map=(0,)
)


def _bcast_lane(v, l):
    return lax.gather(
        v, jnp.full((16, 1), l, jnp.int32), _GATHER_DNUMS, (1,),
        mode=lax.GatherScatterMode.PROMISE_IN_BOUNDS,
    )


@functools.partial(
    pl.kernel,
    out_type=jax.ShapeDtypeStruct((NC, NPAD), jnp.float32),
    mesh=_mesh(),
    compiler_params=_SC_PARAMS,
    scratch_types=[
        pltpu.VMEM((CPW, CHUNK), jnp.int32),
        pltpu.VMEM((CPW, CHUNK), jnp.float32),
        pltpu.VMEM((NPS,), jnp.float32),
        pltpu.VMEM_SHARED((NPAD,), jnp.float32),
    ],
)
def _sc_deg(dst_hbm, ew_hbm, out_hbm, dst_v, ew_v, zb, acc):
    cid = lax.axis_index("c")
    sid = lax.axis_index("s")
    wid = cid * NS + sid

    def zero_zb(i, carry):
        zb[pl.ds(i * 16, 16)] = jnp.zeros((16,), jnp.float32)
        return carry

    lax.fori_loop(0, NPS // 16, zero_zb, 0)
    pltpu.sync_copy(zb, acc.at[pl.ds(sid * NPS, NPS)])
    plsc.subcore_barrier()

    pltpu.sync_copy(dst_hbm.at[wid], dst_v)
    pltpu.sync_copy(ew_hbm.at[wid], ew_v)

    def chunk(j, carry):
        pltpu.sync_copy(ew_v.at[j], acc.at[dst_v.at[j]], add=True)
        return carry

    lax.fori_loop(0, CPW, chunk, 0)
    plsc.subcore_barrier()
    pltpu.sync_copy(acc.at[pl.ds(sid * NPS, NPS)], zb)
    pltpu.sync_copy(zb, out_hbm.at[cid, pl.ds(sid * NPS, NPS)])


def _make_agg(D):
    @functools.partial(
        pl.kernel,
        out_type=jax.ShapeDtypeStruct((NC, NPAD, D), jnp.float32),
        mesh=_mesh(),
        compiler_params=_SC_PARAMS,
        scratch_types=[
            pltpu.VMEM((CPW, CHUNK), jnp.int32),
            pltpu.VMEM((CPW, CHUNK), jnp.int32),
            pltpu.VMEM((CPW, CHUNK), jnp.float32),
            pltpu.VMEM((NPAD,), jnp.float32),
            pltpu.VMEM((CHUNK, D), jnp.float32),
            pltpu.VMEM((CHUNK, D), jnp.float32),
            pltpu.VMEM((CHUNK, D), jnp.float32),
            pltpu.VMEM((CHUNK, D), jnp.float32),
            pltpu.VMEM((64, D), jnp.float32),
            pltpu.VMEM_SHARED((NPAD, D), jnp.float32),
            pltpu.SemaphoreType.DMA,
            pltpu.SemaphoreType.DMA,
            pltpu.SemaphoreType.DMA,
            pltpu.SemaphoreType.DMA,
            pltpu.SemaphoreType.DMA,
            pltpu.SemaphoreType.DMA,
            pltpu.SemaphoreType.DMA,
            pltpu.SemaphoreType.DMA,
        ],
    )
    def agg(h_hbm, src_hbm, dst_hbm, ew_hbm, dis_hbm, out_hbm,
            src_v, dst_v, ew_v, dis_v, r0, r1, r2, r3, zb, acc,
            g0, g1, g2, g3, s0, s1, s2, s3):
        cid = lax.axis_index("c")
        sid = lax.axis_index("s")
        wid = cid * NS + sid
        rows = [r0, r1, r2, r3]
        semg = [g0, g1, g2, g3]
        sems = [s0, s1, s2, s3]

        def zero_zb(r, carry):
            for c in range(D // 16):
                zb[r, pl.ds(c * 16, 16)] = jnp.zeros((16,), jnp.float32)
            return carry

        lax.fori_loop(0, 64, zero_zb, 0)

        def zero_acc(t, carry):
            pltpu.sync_copy(zb, acc.at[pl.ds(sid * NPS + t * 64, 64)])
            return carry

        lax.fori_loop(0, NPS // 64, zero_acc, 0)
        plsc.subcore_barrier()

        pltpu.sync_copy(src_hbm.at[wid], src_v)
        pltpu.sync_copy(dst_hbm.at[wid], dst_v)
        pltpu.sync_copy(ew_hbm.at[wid], ew_v)
        pltpu.sync_copy(dis_hbm, dis_v)

        def gather_start(j, k):
            pltpu.async_copy(h_hbm.at[src_v.at[j]], rows[k], semg[k])

        def gather_wait(j, k):
            pltpu.make_async_copy(h_hbm.at[src_v.at[j]], rows[k], semg[k]).wait()

        def scat_start(j, k):
            pltpu.async_copy(rows[k], acc.at[dst_v.at[j]], sems[k], add=True)

        def scat_wait(j, k):
            pltpu.make_async_copy(rows[k], acc.at[dst_v.at[j]], sems[k]).wait()

        def scale(j, k):
            norms = []
            for g in range(CHUNK // 16):
                si = src_v[j, pl.ds(g * 16, 16)]
                di = dst_v[j, pl.ds(g * 16, 16)]
                ev = ew_v[j, pl.ds(g * 16, 16)]
                norms.append(
                    plsc.load_gather(dis_v, [si]) * ev * plsc.load_gather(dis_v, [di])
                )
            for g in range(CHUNK // 16):
                for l in range(16):
                    r = g * 16 + l
                    b = _bcast_lane(norms[g], l)
                    for c in range(D // 16):
                        rows[k][r, pl.ds(c * 16, 16)] = (
                            rows[k][r, pl.ds(c * 16, 16)] * b
                        )

        gather_start(0, 0)
        gather_start(1, 1)

        NT = CPW // 4

        def body(t, carry):
            for k in range(4):
                c = t * 4 + k
                gather_wait(c, k)
                kn = (k + 2) % 4
                if k < 2:
                    @pl.when(t > 0)
                    def _():
                        scat_wait(c - 2, kn)

                    gather_start(c + 2, kn)
                else:
                    scat_wait(c - 2, kn)

                    @pl.when(t < NT - 1)
                    def _():
                        gather_start(c + 2, kn)
                scale(c, k)
                scat_start(c, k)
            return carry

        lax.fori_loop(0, NT, body, 0)
        scat_wait(CPW - 2, 2)
        scat_wait(CPW - 1, 3)
        plsc.subcore_barrier()

        def writeout(t, carry):
            pltpu.sync_copy(acc.at[pl.ds(sid * NPS + t * 64, 64)], zb)
            pltpu.sync_copy(zb, out_hbm.at[cid, pl.ds(sid * NPS + t * 64, 64)])
            return carry

        lax.fori_loop(0, NPS // 64, writeout, 0)

    return agg


_sc_agg64 = _make_agg(64)
_sc_agg32 = _make_agg(32)


def _mm1_body(x_ref, w_ref, b_ref, o_ref):
    o_ref[...] = (
        jnp.dot(x_ref[...], w_ref[...], preferred_element_type=jnp.float32, precision=lax.Precision.HIGHEST)
        + b_ref[...]
    )


def _dis_body(d_ref, o_ref):
    deg = d_ref[0:1, :] + d_ref[1:2, :] + 1.0
    o_ref[...] = jnp.where(deg > 0, lax.rsqrt(deg), 0.0)


def _l2_body(a_ref, h1_ref, dc_ref, w2_ref, b1_ref, o_ref):
    s = dc_ref[...]
    out1 = a_ref[0] + a_ref[1] + s * s * h1_ref[...] + b1_ref[...]
    out1 = jnp.maximum(out1, 0.0)
    o_ref[...] = jnp.dot(out1, w2_ref[...], preferred_element_type=jnp.float32, precision=lax.Precision.HIGHEST)


def _zadj_body(ai_ref, hi_ref, di_ref, aj_ref, hj_ref, dj_ref, b2_ref,
               adj_ref, z_ref):
    j = pl.program_id(1)
    si = di_ref[...]
    zi = ai_ref[0] + ai_ref[1] + si * si * hi_ref[...] + b2_ref[...]
    sj = dj_ref[...]
    zj = aj_ref[0] + aj_ref[1] + sj * sj * hj_ref[...] + b2_ref[...]
    adj_ref[...] = lax.dot_general(
        zi, zj, (((1,), (1,)), ((), ())), preferred_element_type=jnp.float32
    )

    @pl.when(j == 0)
    def _():
        z_ref[...] = zi


def kernel(x, edge_index, edge_weight, W1, b1, W2, b2):
    src = edge_index[0].astype(jnp.int32)
    dst = edge_index[1].astype(jnp.int32)
    ew = edge_weight.astype(jnp.float32)
    pad = NW * EPW - E
    pad_idx = jnp.arange(pad, dtype=jnp.int32) % N
    src3 = jnp.concatenate([src, pad_idx]).reshape(NW, CPW, CHUNK)
    dst3 = jnp.concatenate([dst, pad_idx]).reshape(NW, CPW, CHUNK)
    ew3 = jnp.concatenate([ew, jnp.zeros((pad,), jnp.float32)]).reshape(
        NW, CPW, CHUNK
    )
    b1r = b1.reshape(1, 64)
    b2r = b2.reshape(1, 32)

    deg2 = _sc_deg(dst3, ew3)

    h1 = pl.pallas_call(
        _mm1_body,
        grid=(10,),
        in_specs=[
            pl.BlockSpec((1000, 128), lambda i: (i, 0)),
            pl.BlockSpec((128, 64), lambda i: (0, 0)),
            pl.BlockSpec((1, 64), lambda i: (0, 0)),
        ],
        out_specs=pl.BlockSpec((1000, 64), lambda i: (i, 0)),
        out_shape=jax.ShapeDtypeStruct((N, 64), jnp.float32),
    )(x, W1, b1r)

    dis2d = pl.pallas_call(
        _dis_body,
        out_shape=jax.ShapeDtypeStruct((1, NPAD), jnp.float32),
    )(deg2)
    dis_flat = dis2d.reshape(NPAD)
    dis_col = dis2d[0, :N].reshape(N, 1)

    agg1 = _sc_agg64(h1, src3, dst3, ew3, dis_flat)

    h2 = pl.pallas_call(
        _l2_body,
        grid=(10,),
        in_specs=[
            pl.BlockSpec((NC, 1000, 64), lambda i: (0, i, 0)),
            pl.BlockSpec((1000, 64), lambda i: (i, 0)),
            pl.BlockSpec((1000, 1), lambda i: (i, 0)),
            pl.BlockSpec((64, 32), lambda i: (0, 0)),
            pl.BlockSpec((1, 64), lambda i: (0, 0)),
        ],
        out_specs=pl.BlockSpec((1000, 32), lambda i: (i, 0)),
        out_shape=jax.ShapeDtypeStruct((N, 32), jnp.float32),
    )(agg1, h1, dis_col, W2, b1r)

    agg2 = _sc_agg32(h2, src3, dst3, ew3, dis_flat)

    adj, z = pl.pallas_call(
        _zadj_body,
        grid=(5, 8),
        in_specs=[
            pl.BlockSpec((NC, 2000, 32), lambda i, j: (0, i, 0)),
            pl.BlockSpec((2000, 32), lambda i, j: (i, 0)),
            pl.BlockSpec((2000, 1), lambda i, j: (i, 0)),
            pl.BlockSpec((NC, 1280, 32), lambda i, j: (0, j, 0)),
            pl.BlockSpec((1280, 32), lambda i, j: (j, 0)),
            pl.BlockSpec((1280, 1), lambda i, j: (j, 0)),
            pl.BlockSpec((1, 32), lambda i, j: (0, 0)),
        ],
        out_specs=[
            pl.BlockSpec((2000, 1280), lambda i, j: (i, j)),
            pl.BlockSpec((2000, 32), lambda i, j: (i, 0)),
        ],
        out_shape=[
            jax.ShapeDtypeStruct((N, N), jnp.float32),
            jax.ShapeDtypeStruct((N, 32), jnp.float32),
        ],
    )(agg2, h2, dis_col, agg2, h2, dis_col, b2r)

    return (adj, z)


# speedup vs baseline: 1.0785x; 1.0548x over previous
import functools

import jax
import jax.numpy as jnp
from jax import lax
from jax.experimental import pallas as pl
from jax.experimental.pallas import tpu as pltpu
from jax.experimental.pallas import tpu_sc as plsc

N = 10000
E = 320000
NC = 2
NS = 16
NW = NC * NS
CHUNK = 128
CPW = 80
EPW = CPW * CHUNK
EPW_REAL = E // NW
NPS = 640
NPAD = NS * NPS


def _mesh():
    return plsc.VectorSubcoreMesh(
        core_axis_name="c", subcore_axis_name="s", num_cores=NC, num_subcores=NS
    )


_SC_PARAMS = pltpu.CompilerParams(
    needs_layout_passes=False, use_tc_tiling_on_sc=False
)

_GATHER_DNUMS = lax.GatherDimensionNumbers(
    offset_dims=(), collapsed_slice_dims=(0,), start_index_map=(0,)
)


def _bcast_lane(v, l):
    return lax.gather(
        v, jnp.full((16, 1), l, jnp.int32), _GATHER_DNUMS, (1,),
        mode=lax.GatherScatterMode.PROMISE_IN_BOUNDS,
    )


@functools.partial(
    pl.kernel,
    out_type=jax.ShapeDtypeStruct((NC, NPAD), jnp.float32),
    mesh=_mesh(),
    compiler_params=_SC_PARAMS,
    scratch_types=[
        pltpu.VMEM((CPW, CHUNK), jnp.int32),
        pltpu.VMEM((CPW, CHUNK), jnp.float32),
        pltpu.VMEM((NPS,), jnp.float32),
        pltpu.VMEM_SHARED((NPAD,), jnp.float32),
    ],
)
def _sc_deg(dst_hbm, ew_hbm, out_hbm, dst_v, ew_v, zb, acc):
    cid = lax.axis_index("c")
    sid = lax.axis_index("s")
    wid = cid * NS + sid

    def zero_zb(i, carry):
        zb[pl.ds(i * 16, 16)] = jnp.zeros((16,), jnp.float32)
        return carry

    lax.fori_loop(0, NPS // 16, zero_zb, 0)
    pltpu.sync_copy(zb, acc.at[pl.ds(sid * NPS, NPS)])
    plsc.subcore_barrier()

    pltpu.sync_copy(dst_hbm.at[wid], dst_v)
    pltpu.sync_copy(ew_hbm.at[wid], ew_v)

    def chunk(j, carry):
        pltpu.sync_copy(ew_v.at[j], acc.at[dst_v.at[j]], add=True)
        return carry

    lax.fori_loop(0, CPW, chunk, 0)
    plsc.subcore_barrier()
    pltpu.sync_copy(acc.at[pl.ds(sid * NPS, NPS)], zb)
    pltpu.sync_copy(zb, out_hbm.at[cid, pl.ds(sid * NPS, NPS)])


def _make_agg(D):
    @functools.partial(
        pl.kernel,
        out_type=jax.ShapeDtypeStruct((NC, NPAD, D), jnp.float32),
        mesh=_mesh(),
        compiler_params=_SC_PARAMS,
        scratch_types=[
            pltpu.VMEM((CPW, CHUNK), jnp.int32),
            pltpu.VMEM((CPW, CHUNK), jnp.int32),
            pltpu.VMEM((CPW, CHUNK), jnp.float32),
            pltpu.VMEM((NPAD,), jnp.float32),
            pltpu.VMEM((CHUNK, D), jnp.float32),
            pltpu.VMEM((CHUNK, D), jnp.float32),
            pltpu.VMEM((CHUNK, D), jnp.float32),
            pltpu.VMEM((CHUNK, D), jnp.float32),
            pltpu.VMEM((64, D), jnp.float32),
            pltpu.VMEM_SHARED((NPAD, D), jnp.float32),
            pltpu.SemaphoreType.DMA,
            pltpu.SemaphoreType.DMA,
            pltpu.SemaphoreType.DMA,
            pltpu.SemaphoreType.DMA,
            pltpu.SemaphoreType.DMA,
            pltpu.SemaphoreType.DMA,
            pltpu.SemaphoreType.DMA,
            pltpu.SemaphoreType.DMA,
        ],
    )
    def agg(h_hbm, src_hbm, dst_hbm, ew_hbm, dis_hbm, out_hbm,
            src_v, dst_v, ew_v, dis_v, r0, r1, r2, r3, zb, acc,
            g0, g1, g2, g3, s0, s1, s2, s3):
        cid = lax.axis_index("c")
        sid = lax.axis_index("s")
        wid = cid * NS + sid
        rows = [r0, r1, r2, r3]
        semg = [g0, g1, g2, g3]
        sems = [s0, s1, s2, s3]

        def zero_zb(r, carry):
            for c in range(D // 16):
                zb[r, pl.ds(c * 16, 16)] = jnp.zeros((16,), jnp.float32)
            return carry

        lax.fori_loop(0, 64, zero_zb, 0)

        def zero_acc(t, carry):
            pltpu.sync_copy(zb, acc.at[pl.ds(sid * NPS + t * 64, 64)])
            return carry

        lax.fori_loop(0, NPS // 64, zero_acc, 0)
        plsc.subcore_barrier()

        pltpu.sync_copy(src_hbm.at[wid], src_v)
        pltpu.sync_copy(dst_hbm.at[wid], dst_v)
        pltpu.sync_copy(ew_hbm.at[wid], ew_v)
        pltpu.sync_copy(dis_hbm, dis_v)

        def gather_start(j, k):
            pltpu.async_copy(h_hbm.at[src_v.at[j]], rows[k], semg[k])

        def gather_wait(j, k):
            pltpu.make_async_copy(h_hbm.at[src_v.at[j]], rows[k], semg[k]).wait()

        def scat_start(j, k):
            pltpu.async_copy(rows[k], acc.at[dst_v.at[j]], sems[k], add=True)

        def scat_wait(j, k):
            pltpu.make_async_copy(rows[k], acc.at[dst_v.at[j]], sems[k]).wait()

        def scale(j, k):
            norms = []
            for g in range(CHUNK // 16):
                si = src_v[j, pl.ds(g * 16, 16)]
                di = dst_v[j, pl.ds(g * 16, 16)]
                ev = ew_v[j, pl.ds(g * 16, 16)]
                norms.append(
                    plsc.load_gather(dis_v, [si]) * ev * plsc.load_gather(dis_v, [di])
                )
            for g in range(CHUNK // 16):
                for l in range(16):
                    r = g * 16 + l
                    b = _bcast_lane(norms[g], l)
                    for c in range(D // 16):
                        rows[k][r, pl.ds(c * 16, 16)] = (
                            rows[k][r, pl.ds(c * 16, 16)] * b
                        )

        gather_start(0, 0)
        gather_start(1, 1)

        NT = CPW // 4

        def body(t, carry):
            for k in range(4):
                c = t * 4 + k
                gather_wait(c, k)
                kn = (k + 2) % 4
                if k < 2:
                    @pl.when(t > 0)
                    def _():
                        scat_wait(c - 2, kn)

                    gather_start(c + 2, kn)
                else:
                    scat_wait(c - 2, kn)

                    @pl.when(t < NT - 1)
                    def _():
                        gather_start(c + 2, kn)
                scale(c, k)
                scat_start(c, k)
            return carry

        lax.fori_loop(0, NT, body, 0)
        scat_wait(CPW - 2, 2)
        scat_wait(CPW - 1, 3)
        plsc.subcore_barrier()

        def writeout(t, carry):
            pltpu.sync_copy(acc.at[pl.ds(sid * NPS + t * 64, 64)], zb)
            pltpu.sync_copy(zb, out_hbm.at[cid, pl.ds(sid * NPS + t * 64, 64)])
            return carry

        lax.fori_loop(0, NPS // 64, writeout, 0)

    return agg


_sc_agg64 = _make_agg(64)
_sc_agg32 = _make_agg(32)


def _mm1_body(x_ref, w_ref, b_ref, o_ref):
    o_ref[...] = (
        jnp.dot(x_ref[...], w_ref[...], preferred_element_type=jnp.float32, precision=lax.Precision.HIGHEST)
        + b_ref[...]
    )


def _dis_body(d_ref, o_ref):
    deg = d_ref[0:1, :] + d_ref[1:2, :] + 1.0
    o_ref[...] = jnp.where(deg > 0, lax.rsqrt(deg), 0.0)


def _l2_body(a_ref, h1_ref, dc_ref, w2_ref, b1_ref, o_ref):
    s = dc_ref[...]
    out1 = a_ref[0] + a_ref[1] + s * s * h1_ref[...] + b1_ref[...]
    out1 = jnp.maximum(out1, 0.0)
    o_ref[...] = jnp.dot(out1, w2_ref[...], preferred_element_type=jnp.float32, precision=lax.Precision.HIGHEST)


def _z_body(a_ref, h2_ref, dc_ref, b2_ref, o_ref):
    s = dc_ref[...]
    o_ref[...] = a_ref[0] + a_ref[1] + s * s * h2_ref[...] + b2_ref[...]


def _adj_body(zi_ref, zj_ref, o_ref):
    o_ref[...] = lax.dot_general(
        zi_ref[...], zj_ref[...], (((1,), (1,)), ((), ())),
        preferred_element_type=jnp.float32,
    )


def kernel(x, edge_index, edge_weight, W1, b1, W2, b2):
    src = edge_index[0].astype(jnp.int32)
    dst = edge_index[1].astype(jnp.int32)
    ew = edge_weight.astype(jnp.float32)
    pad = NW * EPW - E
    pad_idx = jnp.arange(pad, dtype=jnp.int32) % N
    src3 = jnp.concatenate([src, pad_idx]).reshape(NW, CPW, CHUNK)
    dst3 = jnp.concatenate([dst, pad_idx]).reshape(NW, CPW, CHUNK)
    ew3 = jnp.concatenate([ew, jnp.zeros((pad,), jnp.float32)]).reshape(
        NW, CPW, CHUNK
    )
    b1r = b1.reshape(1, 64)
    b2r = b2.reshape(1, 32)

    deg2 = _sc_deg(dst3, ew3)

    h1 = pl.pallas_call(
        _mm1_body,
        grid=(10,),
        in_specs=[
            pl.BlockSpec((1000, 128), lambda i: (i, 0)),
            pl.BlockSpec((128, 64), lambda i: (0, 0)),
            pl.BlockSpec((1, 64), lambda i: (0, 0)),
        ],
        out_specs=pl.BlockSpec((1000, 64), lambda i: (i, 0)),
        out_shape=jax.ShapeDtypeStruct((N, 64), jnp.float32),
    )(x, W1, b1r)

    dis2d = pl.pallas_call(
        _dis_body,
        out_shape=jax.ShapeDtypeStruct((1, NPAD), jnp.float32),
    )(deg2)
    dis_flat = dis2d.reshape(NPAD)
    dis_col = dis2d[0, :N].reshape(N, 1)

    agg1 = _sc_agg64(h1, src3, dst3, ew3, dis_flat)

    h2 = pl.pallas_call(
        _l2_body,
        grid=(10,),
        in_specs=[
            pl.BlockSpec((NC, 1000, 64), lambda i: (0, i, 0)),
            pl.BlockSpec((1000, 64), lambda i: (i, 0)),
            pl.BlockSpec((1000, 1), lambda i: (i, 0)),
            pl.BlockSpec((64, 32), lambda i: (0, 0)),
            pl.BlockSpec((1, 64), lambda i: (0, 0)),
        ],
        out_specs=pl.BlockSpec((1000, 32), lambda i: (i, 0)),
        out_shape=jax.ShapeDtypeStruct((N, 32), jnp.float32),
    )(agg1, h1, dis_col, W2, b1r)

    agg2 = _sc_agg32(h2, src3, dst3, ew3, dis_flat)

    z = pl.pallas_call(
        _z_body,
        grid=(10,),
        in_specs=[
            pl.BlockSpec((NC, 1000, 32), lambda i: (0, i, 0)),
            pl.BlockSpec((1000, 32), lambda i: (i, 0)),
            pl.BlockSpec((1000, 1), lambda i: (i, 0)),
            pl.BlockSpec((1, 32), lambda i: (0, 0)),
        ],
        out_specs=pl.BlockSpec((1000, 32), lambda i: (i, 0)),
        out_shape=jax.ShapeDtypeStruct((N, 32), jnp.float32),
    )(agg2, h2, dis_col, b2r)

    adj = pl.pallas_call(
        _adj_body,
        grid=(5, 8),
        in_specs=[
            pl.BlockSpec((2000, 32), lambda i, j: (i, 0)),
            pl.BlockSpec((1280, 32), lambda i, j: (j, 0)),
        ],
        out_specs=pl.BlockSpec((2000, 1280), lambda i, j: (i, j)),
        out_shape=jax.ShapeDtypeStruct((N, N), jnp.float32),
    )(z, z)

    return (adj, z)
